# Initial kernel scaffold; baseline (speedup 1.0000x reference)
import jax
import jax.numpy as jnp
from jax.experimental import pallas as pl


def kernel(bboxes, scores):
    raise NotImplementedError("write your pallas kernel here")


# speedup vs baseline: 134.9352x; 134.9352x over previous
import jax
import jax.numpy as jnp
from jax.experimental import pallas as pl
from jax.experimental.pallas import tpu as pltpu

_T = 128
_IOU_THR = 0.5


def _col_boxes(bc_tile):
    cz, cy, cx = bc_tile[0:1], bc_tile[1:2], bc_tile[2:3]
    sz, sy, sx = bc_tile[3:4], bc_tile[4:5], bc_tile[5:6]
    lo = (cz - sz / 2.0, cy - sy / 2.0, cx - sx / 2.0)
    hi = (cz + sz / 2.0, cy + sy / 2.0, cx + sx / 2.0)
    vol = (sz * sy) * sx
    return lo, hi, vol


def _row_boxes(bt_tile):
    cz, cy, cx = bt_tile[:, 0:1], bt_tile[:, 1:2], bt_tile[:, 2:3]
    sz, sy, sx = bt_tile[:, 3:4], bt_tile[:, 4:5], bt_tile[:, 5:6]
    lo = (cz - sz / 2.0, cy - sy / 2.0, cx - sx / 2.0)
    hi = (cz + sz / 2.0, cy + sy / 2.0, cx + sx / 2.0)
    vol = (sz * sy) * sx
    return lo, hi, vol


def _iou_tile(rlo, rhi, rvol, clo, chi, cvol):
    o0 = jnp.maximum(jnp.minimum(rhi[0], chi[0]) - jnp.maximum(rlo[0], clo[0]), 0.0)
    o1 = jnp.maximum(jnp.minimum(rhi[1], chi[1]) - jnp.maximum(rlo[1], clo[1]), 0.0)
    o2 = jnp.maximum(jnp.minimum(rhi[2], chi[2]) - jnp.maximum(rlo[2], clo[2]), 0.0)
    inter = (o0 * o1) * o2
    union = (rvol + cvol) - inter
    return inter / union


def _matvec(act, supf):
    return jax.lax.dot_general(
        act, supf, (((1,), (0,)), ((), ())), preferred_element_type=jnp.float32
    )


def _nms_kernel(bc_ref, bt_ref, sc_ref, outs_ref, outb_ref, keep_ref):
    nb = bc_ref.shape[0]
    keep_ref[...] = jnp.ones_like(keep_ref)

    def blk_body(blk, _):
        rlo, rhi, rvol = _row_boxes(bt_ref[blk])
        clo, chi, cvol = _col_boxes(bc_ref[blk])
        iou = _iou_tile(rlo, rhi, rvol, clo, chi, cvol)
        rid = jax.lax.broadcasted_iota(jnp.int32, (_T, _T), 0)
        cid = jax.lax.broadcasted_iota(jnp.int32, (_T, _T), 1)
        supf = jnp.where((iou >= _IOU_THR) & (cid > rid), 1.0, 0.0)
        kblk = keep_ref[blk]

        def wcond(st):
            return st[1] > 0.0

        def wbody(st):
            act, _ = st
            sup = _matvec(act, supf)
            new = jnp.where(sup > 0.5, 0.0, kblk)
            changed = jnp.sum(jnp.abs(new - act))
            return new, changed

        act, _ = jax.lax.while_loop(wcond, wbody, (kblk, jnp.float32(1.0)))
        keep_ref[blk] = act

        def w_body(w, _):
            clo2, chi2, cvol2 = _col_boxes(bc_ref[w])
            iou2 = _iou_tile(rlo, rhi, rvol, clo2, chi2, cvol2)
            supf2 = jnp.where(iou2 >= _IOU_THR, 1.0, 0.0)
            sup2 = _matvec(act, supf2)
            keep_ref[w] = jnp.where(sup2 > 0.5, 0.0, keep_ref[w])
            return ()

        jax.lax.fori_loop(blk + 1, nb, w_body, ())
        return ()

    jax.lax.fori_loop(0, nb, blk_body, ())

    keep = keep_ref[...]
    outs_ref[...] = sc_ref[...] * keep
    outb_ref[...] = bc_ref[...] * keep


def kernel(bboxes, scores):
    bat, _, n = bboxes.shape
    npad = ((n + _T - 1) // _T) * _T
    nb = npad // _T

    order = jnp.argsort(-scores, axis=1)
    b = jnp.transpose(bboxes, (0, 2, 1))
    b_sorted = jnp.take_along_axis(b, order[:, :, None], axis=1)
    sc_sorted = jnp.take_along_axis(scores, order, axis=1)

    pad = npad - n
    bp = jnp.pad(b_sorted, ((0, 0), (0, pad), (0, 0)))
    scp = jnp.pad(sc_sorted, ((0, 0), (0, pad)))

    bt3 = bp.reshape(bat, nb, _T, 6)
    bc3 = jnp.transpose(bt3, (0, 1, 3, 2))
    sc3 = scp.reshape(bat, nb, 1, _T)

    outs, outb = pl.pallas_call(
        _nms_kernel,
        grid=(bat,),
        in_specs=[
            pl.BlockSpec((None, nb, 6, _T), lambda i: (i, 0, 0, 0)),
            pl.BlockSpec((None, nb, _T, 6), lambda i: (i, 0, 0, 0)),
            pl.BlockSpec((None, nb, 1, _T), lambda i: (i, 0, 0, 0)),
        ],
        out_specs=[
            pl.BlockSpec((None, nb, 1, _T), lambda i: (i, 0, 0, 0)),
            pl.BlockSpec((None, nb, 6, _T), lambda i: (i, 0, 0, 0)),
        ],
        out_shape=[
            jax.ShapeDtypeStruct((bat, nb, 1, _T), jnp.float32),
            jax.ShapeDtypeStruct((bat, nb, 6, _T), jnp.float32),
        ],
        scratch_shapes=[pltpu.VMEM((nb, 1, _T), jnp.float32)],
    )(bc3, bt3, sc3)

    sel_scores = outs.reshape(bat, npad)[:, :n]
    sel_boxes = jnp.transpose(outb, (0, 1, 3, 2)).reshape(bat, npad, 6)[:, :n, :]
    return jnp.concatenate([sel_scores[:, :, None], sel_boxes], axis=2)


# speedup vs baseline: 192.0962x; 1.4236x over previous
import jax
import jax.numpy as jnp
from jax.experimental import pallas as pl
from jax.experimental.pallas import tpu as pltpu

_T = 128
_IOU_THR = 0.5


def _col_boxes(bc_tile):
    cz, cy, cx = bc_tile[0:1], bc_tile[1:2], bc_tile[2:3]
    sz, sy, sx = bc_tile[3:4], bc_tile[4:5], bc_tile[5:6]
    lo = (cz - sz / 2.0, cy - sy / 2.0, cx - sx / 2.0)
    hi = (cz + sz / 2.0, cy + sy / 2.0, cx + sx / 2.0)
    vol = (sz * sy) * sx
    return lo, hi, vol


def _row_boxes(bt_tile):
    cz, cy, cx = bt_tile[:, 0:1], bt_tile[:, 1:2], bt_tile[:, 2:3]
    sz, sy, sx = bt_tile[:, 3:4], bt_tile[:, 4:5], bt_tile[:, 5:6]
    lo = (cz - sz / 2.0, cy - sy / 2.0, cx - sx / 2.0)
    hi = (cz + sz / 2.0, cy + sy / 2.0, cx + sx / 2.0)
    vol = (sz * sy) * sx
    return lo, hi, vol


def _iou_tile(rows, cols):
    (rlo, rhi, rvol), (clo, chi, cvol) = rows, cols
    o0 = jnp.maximum(jnp.minimum(rhi[0], chi[0]) - jnp.maximum(rlo[0], clo[0]), 0.0)
    o1 = jnp.maximum(jnp.minimum(rhi[1], chi[1]) - jnp.maximum(rlo[1], clo[1]), 0.0)
    o2 = jnp.maximum(jnp.minimum(rhi[2], chi[2]) - jnp.maximum(rlo[2], clo[2]), 0.0)
    inter = (o0 * o1) * o2
    union = (rvol + cvol) - inter
    return inter / union


def _matvec(act, supf):
    return jax.lax.dot_general(
        act, supf, (((1,), (0,)), ((), ())), preferred_element_type=jnp.float32
    )


def _nms_kernel(bc_ref, bt_ref, sc_ref, outs_ref, outb_ref, keep_ref):
    nb = bc_ref.shape[1]
    keep_ref[...] = jnp.ones_like(keep_ref)

    def blk_body(blk, _):
        rows0 = _row_boxes(bt_ref[0, blk])
        rows1 = _row_boxes(bt_ref[1, blk])
        iou0 = _iou_tile(rows0, _col_boxes(bc_ref[0, blk]))
        iou1 = _iou_tile(rows1, _col_boxes(bc_ref[1, blk]))
        rid = jax.lax.broadcasted_iota(jnp.int32, (_T, _T), 0)
        cid = jax.lax.broadcasted_iota(jnp.int32, (_T, _T), 1)
        tri = cid > rid
        supf0 = jnp.where((iou0 >= _IOU_THR) & tri, 1.0, 0.0)
        supf1 = jnp.where((iou1 >= _IOU_THR) & tri, 1.0, 0.0)
        kblk0 = keep_ref[0, blk]
        kblk1 = keep_ref[1, blk]

        def wcond(st):
            return st[2] > 0.0

        def wbody(st):
            act0, act1, _ = st
            new0 = jnp.where(_matvec(act0, supf0) > 0.5, 0.0, kblk0)
            new1 = jnp.where(_matvec(act1, supf1) > 0.5, 0.0, kblk1)
            changed = jnp.sum(jnp.abs(new0 - act0)) + jnp.sum(jnp.abs(new1 - act1))
            return new0, new1, changed

        act0, act1, _ = jax.lax.while_loop(
            wcond, wbody, (kblk0, kblk1, jnp.float32(1.0))
        )
        keep_ref[0, blk] = act0
        keep_ref[1, blk] = act1

        def w_body(w, _):
            s0 = jnp.where(_iou_tile(rows0, _col_boxes(bc_ref[0, w])) >= _IOU_THR, 1.0, 0.0)
            s1 = jnp.where(_iou_tile(rows1, _col_boxes(bc_ref[1, w])) >= _IOU_THR, 1.0, 0.0)
            sup0 = _matvec(act0, s0)
            sup1 = _matvec(act1, s1)
            keep_ref[0, w] = jnp.where(sup0 > 0.5, 0.0, keep_ref[0, w])
            keep_ref[1, w] = jnp.where(sup1 > 0.5, 0.0, keep_ref[1, w])
            return ()

        jax.lax.fori_loop(blk + 1, nb, w_body, ())
        return ()

    jax.lax.fori_loop(0, nb, blk_body, ())

    keep = keep_ref[...]
    outs_ref[...] = sc_ref[...] * keep
    outb_ref[...] = bc_ref[...] * keep


def kernel(bboxes, scores):
    bat, _, n = bboxes.shape
    npad = ((n + _T - 1) // _T) * _T
    nb = npad // _T

    order = jnp.argsort(-scores, axis=1)
    b = jnp.transpose(bboxes, (0, 2, 1))
    b_sorted = jnp.take_along_axis(b, order[:, :, None], axis=1)
    sc_sorted = jnp.take_along_axis(scores, order, axis=1)

    pad = npad - n
    bp = jnp.pad(b_sorted, ((0, 0), (0, pad), (0, 0)))
    scp = jnp.pad(sc_sorted, ((0, 0), (0, pad)))

    bt3 = bp.reshape(bat, nb, _T, 6)
    bc3 = jnp.transpose(bt3, (0, 1, 3, 2))
    sc3 = scp.reshape(bat, nb, 1, _T)

    outs, outb = pl.pallas_call(
        _nms_kernel,
        grid=(1,),
        in_specs=[
            pl.BlockSpec((bat, nb, 6, _T), lambda i: (0, 0, 0, 0)),
            pl.BlockSpec((bat, nb, _T, 6), lambda i: (0, 0, 0, 0)),
            pl.BlockSpec((bat, nb, 1, _T), lambda i: (0, 0, 0, 0)),
        ],
        out_specs=[
            pl.BlockSpec((bat, nb, 1, _T), lambda i: (0, 0, 0, 0)),
            pl.BlockSpec((bat, nb, 6, _T), lambda i: (0, 0, 0, 0)),
        ],
        out_shape=[
            jax.ShapeDtypeStruct((bat, nb, 1, _T), jnp.float32),
            jax.ShapeDtypeStruct((bat, nb, 6, _T), jnp.float32),
        ],
        scratch_shapes=[pltpu.VMEM((bat, nb, 1, _T), jnp.float32)],
    )(bc3, bt3, sc3)

    sel_scores = outs.reshape(bat, npad)[:, :n]
    sel_boxes = jnp.transpose(outb, (0, 1, 3, 2)).reshape(bat, npad, 6)[:, :n, :]
    return jnp.concatenate([sel_scores[:, :, None], sel_boxes], axis=2)


# speedup vs baseline: 231.1050x; 1.2031x over previous
import jax
import jax.numpy as jnp
from jax.experimental import pallas as pl
from jax.experimental.pallas import tpu as pltpu

_T = 128
_IOU_THR = 0.5


def _col_boxes(bc_tile):
    cz, cy, cx = bc_tile[0:1], bc_tile[1:2], bc_tile[2:3]
    sz, sy, sx = bc_tile[3:4], bc_tile[4:5], bc_tile[5:6]
    lo = (cz - sz / 2.0, cy - sy / 2.0, cx - sx / 2.0)
    hi = (cz + sz / 2.0, cy + sy / 2.0, cx + sx / 2.0)
    vol = (sz * sy) * sx
    return lo, hi, vol


def _row_boxes(bt_tile):
    cz, cy, cx = bt_tile[:, 0:1], bt_tile[:, 1:2], bt_tile[:, 2:3]
    sz, sy, sx = bt_tile[:, 3:4], bt_tile[:, 4:5], bt_tile[:, 5:6]
    lo = (cz - sz / 2.0, cy - sy / 2.0, cx - sx / 2.0)
    hi = (cz + sz / 2.0, cy + sy / 2.0, cx + sx / 2.0)
    vol = (sz * sy) * sx
    return lo, hi, vol


def _iou_tile(rows, cols):
    (rlo, rhi, rvol), (clo, chi, cvol) = rows, cols
    o0 = jnp.maximum(jnp.minimum(rhi[0], chi[0]) - jnp.maximum(rlo[0], clo[0]), 0.0)
    o1 = jnp.maximum(jnp.minimum(rhi[1], chi[1]) - jnp.maximum(rlo[1], clo[1]), 0.0)
    o2 = jnp.maximum(jnp.minimum(rhi[2], chi[2]) - jnp.maximum(rlo[2], clo[2]), 0.0)
    inter = (o0 * o1) * o2
    union = (rvol + cvol) - inter
    return inter / union


def _matvec(act, supf):
    return jax.lax.dot_general(
        act, supf, (((1,), (0,)), ((), ())), preferred_element_type=jnp.float32
    )


def _nms_kernel(bc_ref, bt_ref, sc_ref, outs_ref, outb_ref, keep_ref):
    nb = bc_ref.shape[1]
    keep_ref[...] = jnp.ones_like(keep_ref)

    def blk_body(blk, _):
        rows0 = _row_boxes(bt_ref[0, blk])
        rows1 = _row_boxes(bt_ref[1, blk])
        iou0 = _iou_tile(rows0, _col_boxes(bc_ref[0, blk]))
        iou1 = _iou_tile(rows1, _col_boxes(bc_ref[1, blk]))
        rid = jax.lax.broadcasted_iota(jnp.int32, (_T, _T), 0)
        cid = jax.lax.broadcasted_iota(jnp.int32, (_T, _T), 1)
        tri = cid > rid
        supf0 = jnp.where((iou0 >= _IOU_THR) & tri, 1.0, 0.0)
        supf1 = jnp.where((iou1 >= _IOU_THR) & tri, 1.0, 0.0)
        kblk0 = keep_ref[0, blk]
        kblk1 = keep_ref[1, blk]

        def wcond(st):
            return st[2] > 0.0

        def wbody(st):
            act0, act1, _ = st
            new0 = jnp.where(_matvec(act0, supf0) > 0.5, 0.0, kblk0)
            new1 = jnp.where(_matvec(act1, supf1) > 0.5, 0.0, kblk1)
            changed = jnp.sum(jnp.abs(new0 - act0)) + jnp.sum(jnp.abs(new1 - act1))
            return new0, new1, changed

        act0, act1, _ = jax.lax.while_loop(
            wcond, wbody, (kblk0, kblk1, jnp.float32(1.0))
        )
        keep_ref[0, blk] = act0
        keep_ref[1, blk] = act1

        def suppress_at(w):
            s0 = jnp.where(_iou_tile(rows0, _col_boxes(bc_ref[0, w])) >= _IOU_THR, 1.0, 0.0)
            s1 = jnp.where(_iou_tile(rows1, _col_boxes(bc_ref[1, w])) >= _IOU_THR, 1.0, 0.0)
            sup0 = _matvec(act0, s0)
            sup1 = _matvec(act1, s1)
            keep_ref[0, w] = jnp.where(sup0 > 0.5, 0.0, keep_ref[0, w])
            keep_ref[1, w] = jnp.where(sup1 > 0.5, 0.0, keep_ref[1, w])

        nw = nb - blk - 1

        def w2_body(k, _):
            w = blk + 1 + 2 * k
            suppress_at(w)
            suppress_at(w + 1)
            return ()

        jax.lax.fori_loop(0, nw // 2, w2_body, ())

        @pl.when(nw % 2 == 1)
        def _tail():
            suppress_at(nb - 1)

        return ()

    jax.lax.fori_loop(0, nb, blk_body, ())

    keep = keep_ref[...]
    outs_ref[...] = sc_ref[...] * keep
    outb_ref[...] = bc_ref[...] * keep


def kernel(bboxes, scores):
    bat, _, n = bboxes.shape
    npad = ((n + _T - 1) // _T) * _T
    nb = npad // _T

    order = jnp.argsort(-scores, axis=1)
    b = jnp.transpose(bboxes, (0, 2, 1))
    b_sorted = jnp.take_along_axis(b, order[:, :, None], axis=1)
    sc_sorted = jnp.take_along_axis(scores, order, axis=1)

    pad = npad - n
    bp = jnp.pad(b_sorted, ((0, 0), (0, pad), (0, 0)))
    scp = jnp.pad(sc_sorted, ((0, 0), (0, pad)))

    bt3 = bp.reshape(bat, nb, _T, 6)
    bc3 = jnp.transpose(bt3, (0, 1, 3, 2))
    sc3 = scp.reshape(bat, nb, 1, _T)

    outs, outb = pl.pallas_call(
        _nms_kernel,
        grid=(1,),
        in_specs=[
            pl.BlockSpec((bat, nb, 6, _T), lambda i: (0, 0, 0, 0)),
            pl.BlockSpec((bat, nb, _T, 6), lambda i: (0, 0, 0, 0)),
            pl.BlockSpec((bat, nb, 1, _T), lambda i: (0, 0, 0, 0)),
        ],
        out_specs=[
            pl.BlockSpec((bat, nb, 1, _T), lambda i: (0, 0, 0, 0)),
            pl.BlockSpec((bat, nb, 6, _T), lambda i: (0, 0, 0, 0)),
        ],
        out_shape=[
            jax.ShapeDtypeStruct((bat, nb, 1, _T), jnp.float32),
            jax.ShapeDtypeStruct((bat, nb, 6, _T), jnp.float32),
        ],
        scratch_shapes=[pltpu.VMEM((bat, nb, 1, _T), jnp.float32)],
    )(bc3, bt3, sc3)

    sel_scores = outs.reshape(bat, npad)[:, :n]
    sel_boxes = jnp.transpose(outb, (0, 1, 3, 2)).reshape(bat, npad, 6)[:, :n, :]
    return jnp.concatenate([sel_scores[:, :, None], sel_boxes], axis=2)


# speedup vs baseline: 231.9419x; 1.0036x over previous
import jax
import jax.numpy as jnp
from jax.experimental import pallas as pl
from jax.experimental.pallas import tpu as pltpu

_T = 128
_IOU_THR = 0.5


def _col_boxes(p_tile):
    lo = (p_tile[0:1], p_tile[1:2], p_tile[2:3])
    hi = (p_tile[3:4], p_tile[4:5], p_tile[5:6])
    vol = p_tile[6:7]
    return lo, hi, vol


def _row_boxes(bt_tile):
    cz, cy, cx = bt_tile[:, 0:1], bt_tile[:, 1:2], bt_tile[:, 2:3]
    sz, sy, sx = bt_tile[:, 3:4], bt_tile[:, 4:5], bt_tile[:, 5:6]
    lo = (cz - sz / 2.0, cy - sy / 2.0, cx - sx / 2.0)
    hi = (cz + sz / 2.0, cy + sy / 2.0, cx + sx / 2.0)
    vol = (sz * sy) * sx
    return lo, hi, vol


def _iou_tile(rows, cols):
    (rlo, rhi, rvol), (clo, chi, cvol) = rows, cols
    o0 = jnp.maximum(jnp.minimum(rhi[0], chi[0]) - jnp.maximum(rlo[0], clo[0]), 0.0)
    o1 = jnp.maximum(jnp.minimum(rhi[1], chi[1]) - jnp.maximum(rlo[1], clo[1]), 0.0)
    o2 = jnp.maximum(jnp.minimum(rhi[2], chi[2]) - jnp.maximum(rlo[2], clo[2]), 0.0)
    inter = (o0 * o1) * o2
    union = (rvol + cvol) - inter
    return inter / union


def _matvec(act, supf):
    return jax.lax.dot_general(
        act, supf, (((1,), (0,)), ((), ())), preferred_element_type=jnp.float32
    )


def _nms_kernel(bc_ref, bt_ref, sc_ref, outs_ref, outb_ref, keep_ref, colp_ref):
    nb = bc_ref.shape[1]
    keep_ref[...] = jnp.ones_like(keep_ref)

    def pre_body(i, _):
        for b in (0, 1):
            tile = bc_ref[b, i]
            colp_ref[b, i, 0:3] = tile[0:3] - tile[3:6] * 0.5
            colp_ref[b, i, 3:6] = tile[0:3] + tile[3:6] * 0.5
            colp_ref[b, i, 6:7] = (tile[3:4] * tile[4:5]) * tile[5:6]
        return ()

    jax.lax.fori_loop(0, nb, pre_body, ())

    def blk_body(blk, _):
        rows0 = _row_boxes(bt_ref[0, blk])
        rows1 = _row_boxes(bt_ref[1, blk])
        iou0 = _iou_tile(rows0, _col_boxes(colp_ref[0, blk]))
        iou1 = _iou_tile(rows1, _col_boxes(colp_ref[1, blk]))
        rid = jax.lax.broadcasted_iota(jnp.int32, (_T, _T), 0)
        cid = jax.lax.broadcasted_iota(jnp.int32, (_T, _T), 1)
        tri = cid > rid
        supf0 = jnp.where((iou0 >= _IOU_THR) & tri, 1.0, 0.0)
        supf1 = jnp.where((iou1 >= _IOU_THR) & tri, 1.0, 0.0)
        kblk0 = keep_ref[0, blk]
        kblk1 = keep_ref[1, blk]

        def wcond(st):
            return st[2] > 0.0

        def wbody(st):
            act0, act1, _ = st
            new0 = jnp.where(_matvec(act0, supf0) > 0.5, 0.0, kblk0)
            new1 = jnp.where(_matvec(act1, supf1) > 0.5, 0.0, kblk1)
            changed = jnp.sum(jnp.abs(new0 - act0)) + jnp.sum(jnp.abs(new1 - act1))
            return new0, new1, changed

        act0, act1, _ = jax.lax.while_loop(
            wcond, wbody, (kblk0, kblk1, jnp.float32(1.0))
        )
        keep_ref[0, blk] = act0
        keep_ref[1, blk] = act1

        def suppress_at(w):
            s0 = jnp.where(_iou_tile(rows0, _col_boxes(colp_ref[0, w])) >= _IOU_THR, 1.0, 0.0)
            s1 = jnp.where(_iou_tile(rows1, _col_boxes(colp_ref[1, w])) >= _IOU_THR, 1.0, 0.0)
            sup0 = _matvec(act0, s0)
            sup1 = _matvec(act1, s1)
            keep_ref[0, w] = jnp.where(sup0 > 0.5, 0.0, keep_ref[0, w])
            keep_ref[1, w] = jnp.where(sup1 > 0.5, 0.0, keep_ref[1, w])

        nw = nb - blk - 1

        def w2_body(k, _):
            w = blk + 1 + 2 * k
            suppress_at(w)
            suppress_at(w + 1)
            return ()

        jax.lax.fori_loop(0, nw // 2, w2_body, ())

        @pl.when(nw % 2 == 1)
        def _tail():
            suppress_at(nb - 1)

        return ()

    jax.lax.fori_loop(0, nb, blk_body, ())

    keep = keep_ref[...]
    outs_ref[...] = sc_ref[...] * keep
    outb_ref[...] = bc_ref[...] * keep


def kernel(bboxes, scores):
    bat, _, n = bboxes.shape
    npad = ((n + _T - 1) // _T) * _T
    nb = npad // _T

    order = jnp.argsort(-scores, axis=1)
    b = jnp.transpose(bboxes, (0, 2, 1))
    b_sorted = jnp.take_along_axis(b, order[:, :, None], axis=1)
    sc_sorted = jnp.take_along_axis(scores, order, axis=1)

    pad = npad - n
    bp = jnp.pad(b_sorted, ((0, 0), (0, pad), (0, 0)))
    scp = jnp.pad(sc_sorted, ((0, 0), (0, pad)))

    bt3 = bp.reshape(bat, nb, _T, 6)
    bc3 = jnp.transpose(bt3, (0, 1, 3, 2))
    sc3 = scp.reshape(bat, nb, 1, _T)

    outs, outb = pl.pallas_call(
        _nms_kernel,
        grid=(1,),
        in_specs=[
            pl.BlockSpec((bat, nb, 6, _T), lambda i: (0, 0, 0, 0)),
            pl.BlockSpec((bat, nb, _T, 6), lambda i: (0, 0, 0, 0)),
            pl.BlockSpec((bat, nb, 1, _T), lambda i: (0, 0, 0, 0)),
        ],
        out_specs=[
            pl.BlockSpec((bat, nb, 1, _T), lambda i: (0, 0, 0, 0)),
            pl.BlockSpec((bat, nb, 6, _T), lambda i: (0, 0, 0, 0)),
        ],
        out_shape=[
            jax.ShapeDtypeStruct((bat, nb, 1, _T), jnp.float32),
            jax.ShapeDtypeStruct((bat, nb, 6, _T), jnp.float32),
        ],
        scratch_shapes=[
            pltpu.VMEM((bat, nb, 1, _T), jnp.float32),
            pltpu.VMEM((bat, nb, 8, _T), jnp.float32),
        ],
    )(bc3, bt3, sc3)

    sel_scores = outs.reshape(bat, npad)[:, :n]
    sel_boxes = jnp.transpose(outb, (0, 1, 3, 2)).reshape(bat, npad, 6)[:, :n, :]
    return jnp.concatenate([sel_scores[:, :, None], sel_boxes], axis=2)


# speedup vs baseline: 254.6645x; 1.0980x over previous
import jax
import jax.numpy as jnp
from jax.experimental import pallas as pl
from jax.experimental.pallas import tpu as pltpu

_T = 128
_IOU_THR = 0.5


def _col_boxes(p_tile):
    lo = (p_tile[0:1], p_tile[1:2], p_tile[2:3])
    hi = (p_tile[3:4], p_tile[4:5], p_tile[5:6])
    vol = p_tile[6:7]
    return lo, hi, vol


def _row_boxes(bt_tile):
    cz, cy, cx = bt_tile[:, 0:1], bt_tile[:, 1:2], bt_tile[:, 2:3]
    sz, sy, sx = bt_tile[:, 3:4], bt_tile[:, 4:5], bt_tile[:, 5:6]
    lo = (cz - sz / 2.0, cy - sy / 2.0, cx - sx / 2.0)
    hi = (cz + sz / 2.0, cy + sy / 2.0, cx + sx / 2.0)
    vol = (sz * sy) * sx
    return lo, hi, vol


def _iou_tile(rows, cols):
    (rlo, rhi, rvol), (clo, chi, cvol) = rows, cols
    o0 = jnp.maximum(jnp.minimum(rhi[0], chi[0]) - jnp.maximum(rlo[0], clo[0]), 0.0)
    o1 = jnp.maximum(jnp.minimum(rhi[1], chi[1]) - jnp.maximum(rlo[1], clo[1]), 0.0)
    o2 = jnp.maximum(jnp.minimum(rhi[2], chi[2]) - jnp.maximum(rlo[2], clo[2]), 0.0)
    inter = (o0 * o1) * o2
    union = (rvol + cvol) - inter
    return inter / union


def _matvec(act, supf):
    return jax.lax.dot_general(
        act, supf, (((1,), (0,)), ((), ())), preferred_element_type=jnp.float32
    )


def _nms_kernel(bc_ref, bt_ref, sc_ref, outs_ref, outb_ref, keep_ref, colp_ref):
    nb = bc_ref.shape[1]
    keep_ref[...] = jnp.ones_like(keep_ref)

    def pre_body(i, _):
        for b in (0, 1):
            tile = bc_ref[b, i]
            colp_ref[b, i, 0:3] = tile[0:3] - tile[3:6] * 0.5
            colp_ref[b, i, 3:6] = tile[0:3] + tile[3:6] * 0.5
            colp_ref[b, i, 6:7] = (tile[3:4] * tile[4:5]) * tile[5:6]
        return ()

    jax.lax.fori_loop(0, nb, pre_body, ())

    def blk_body(blk, _):
        rows0 = _row_boxes(bt_ref[0, blk])
        rows1 = _row_boxes(bt_ref[1, blk])
        iou0 = _iou_tile(rows0, _col_boxes(colp_ref[0, blk]))
        iou1 = _iou_tile(rows1, _col_boxes(colp_ref[1, blk]))
        rid = jax.lax.broadcasted_iota(jnp.int32, (_T, _T), 0)
        cid = jax.lax.broadcasted_iota(jnp.int32, (_T, _T), 1)
        tri = cid > rid
        supf0 = jnp.where((iou0 >= _IOU_THR) & tri, 1.0, 0.0)
        supf1 = jnp.where((iou1 >= _IOU_THR) & tri, 1.0, 0.0)
        kblk0 = keep_ref[0, blk]
        kblk1 = keep_ref[1, blk]

        def wcond(st):
            return st[2] > 0.0

        def wbody(st):
            act0, act1, _ = st
            new0 = jnp.where(_matvec(act0, supf0) > 0.5, 0.0, kblk0)
            new1 = jnp.where(_matvec(act1, supf1) > 0.5, 0.0, kblk1)
            changed = jnp.sum(jnp.abs(new0 - act0)) + jnp.sum(jnp.abs(new1 - act1))
            return new0, new1, changed

        act0, act1, _ = jax.lax.while_loop(
            wcond, wbody, (kblk0, kblk1, jnp.float32(1.0))
        )
        keep_ref[0, blk] = act0
        keep_ref[1, blk] = act1

        def suppress_at(w):
            s0 = jnp.where(_iou_tile(rows0, _col_boxes(colp_ref[0, w])) >= _IOU_THR, 1.0, 0.0)
            s1 = jnp.where(_iou_tile(rows1, _col_boxes(colp_ref[1, w])) >= _IOU_THR, 1.0, 0.0)
            sup0 = _matvec(act0, s0)
            sup1 = _matvec(act1, s1)
            keep_ref[0, w] = jnp.where(sup0 > 0.5, 0.0, keep_ref[0, w])
            keep_ref[1, w] = jnp.where(sup1 > 0.5, 0.0, keep_ref[1, w])

        nw = nb - blk - 1

        def w4_body(k, _):
            w = blk + 1 + 4 * k
            suppress_at(w)
            suppress_at(w + 1)
            suppress_at(w + 2)
            suppress_at(w + 3)
            return ()

        jax.lax.fori_loop(0, nw // 4, w4_body, ())

        def tail_body(w, _):
            suppress_at(w)
            return ()

        jax.lax.fori_loop(blk + 1 + (nw // 4) * 4, nb, tail_body, ())
        return ()

    jax.lax.fori_loop(0, nb, blk_body, ())

    keep = keep_ref[...]
    outs_ref[...] = sc_ref[...] * keep
    outb_ref[...] = bc_ref[...] * keep


def kernel(bboxes, scores):
    bat, _, n = bboxes.shape
    npad = ((n + _T - 1) // _T) * _T
    nb = npad // _T

    order = jnp.argsort(-scores, axis=1)
    b = jnp.transpose(bboxes, (0, 2, 1))
    b_sorted = jnp.take_along_axis(b, order[:, :, None], axis=1)
    sc_sorted = jnp.take_along_axis(scores, order, axis=1)

    pad = npad - n
    bp = jnp.pad(b_sorted, ((0, 0), (0, pad), (0, 0)))
    scp = jnp.pad(sc_sorted, ((0, 0), (0, pad)))

    bt3 = bp.reshape(bat, nb, _T, 6)
    bc3 = jnp.transpose(bt3, (0, 1, 3, 2))
    sc3 = scp.reshape(bat, nb, 1, _T)

    outs, outb = pl.pallas_call(
        _nms_kernel,
        grid=(1,),
        in_specs=[
            pl.BlockSpec((bat, nb, 6, _T), lambda i: (0, 0, 0, 0)),
            pl.BlockSpec((bat, nb, _T, 6), lambda i: (0, 0, 0, 0)),
            pl.BlockSpec((bat, nb, 1, _T), lambda i: (0, 0, 0, 0)),
        ],
        out_specs=[
            pl.BlockSpec((bat, nb, 1, _T), lambda i: (0, 0, 0, 0)),
            pl.BlockSpec((bat, nb, 6, _T), lambda i: (0, 0, 0, 0)),
        ],
        out_shape=[
            jax.ShapeDtypeStruct((bat, nb, 1, _T), jnp.float32),
            jax.ShapeDtypeStruct((bat, nb, 6, _T), jnp.float32),
        ],
        scratch_shapes=[
            pltpu.VMEM((bat, nb, 1, _T), jnp.float32),
            pltpu.VMEM((bat, nb, 8, _T), jnp.float32),
        ],
    )(bc3, bt3, sc3)

    sel_scores = outs.reshape(bat, npad)[:, :n]
    sel_boxes = jnp.transpose(outb, (0, 1, 3, 2)).reshape(bat, npad, 6)[:, :n, :]
    return jnp.concatenate([sel_scores[:, :, None], sel_boxes], axis=2)


# speedup vs baseline: 256.6026x; 1.0076x over previous
import jax
import jax.numpy as jnp
from jax.experimental import pallas as pl
from jax.experimental.pallas import tpu as pltpu

_T = 128
_IOU_THR = 0.5


def _col_boxes(p_tile):
    lo = (p_tile[0:1], p_tile[1:2], p_tile[2:3])
    hi = (p_tile[3:4], p_tile[4:5], p_tile[5:6])
    vol = p_tile[6:7]
    return lo, hi, vol


def _row_boxes(bt_tile):
    cz, cy, cx = bt_tile[:, 0:1], bt_tile[:, 1:2], bt_tile[:, 2:3]
    sz, sy, sx = bt_tile[:, 3:4], bt_tile[:, 4:5], bt_tile[:, 5:6]
    lo = (cz - sz / 2.0, cy - sy / 2.0, cx - sx / 2.0)
    hi = (cz + sz / 2.0, cy + sy / 2.0, cx + sx / 2.0)
    vol = (sz * sy) * sx
    return lo, hi, vol


def _iou_tile(rows, cols):
    (rlo, rhi, rvol), (clo, chi, cvol) = rows, cols
    o0 = jnp.maximum(jnp.minimum(rhi[0], chi[0]) - jnp.maximum(rlo[0], clo[0]), 0.0)
    o1 = jnp.maximum(jnp.minimum(rhi[1], chi[1]) - jnp.maximum(rlo[1], clo[1]), 0.0)
    o2 = jnp.maximum(jnp.minimum(rhi[2], chi[2]) - jnp.maximum(rlo[2], clo[2]), 0.0)
    inter = (o0 * o1) * o2
    union = (rvol + cvol) - inter
    return inter / union


def _matvec(act, supf):
    return jax.lax.dot_general(
        act, supf, (((1,), (0,)), ((), ())), preferred_element_type=jnp.float32
    )


def _nms_kernel(bc_ref, bt_ref, sc_ref, outs_ref, outb_ref, keep_ref, colp_ref):
    nb = bc_ref.shape[1]
    keep_ref[...] = jnp.ones_like(keep_ref)

    def pre_body(i, _):
        for b in (0, 1):
            tile = bc_ref[b, i]
            colp_ref[b, i, 0:3] = tile[0:3] - tile[3:6] * 0.5
            colp_ref[b, i, 3:6] = tile[0:3] + tile[3:6] * 0.5
            colp_ref[b, i, 6:7] = (tile[3:4] * tile[4:5]) * tile[5:6]
        return ()

    jax.lax.fori_loop(0, nb, pre_body, ())

    def blk_body(blk, _):
        rows0 = _row_boxes(bt_ref[0, blk])
        rows1 = _row_boxes(bt_ref[1, blk])
        iou0 = _iou_tile(rows0, _col_boxes(colp_ref[0, blk]))
        iou1 = _iou_tile(rows1, _col_boxes(colp_ref[1, blk]))
        rid = jax.lax.broadcasted_iota(jnp.int32, (_T, _T), 0)
        cid = jax.lax.broadcasted_iota(jnp.int32, (_T, _T), 1)
        tri = cid > rid
        supf0 = jnp.where((iou0 >= _IOU_THR) & tri, 1.0, 0.0)
        supf1 = jnp.where((iou1 >= _IOU_THR) & tri, 1.0, 0.0)
        kblk0 = keep_ref[0, blk]
        kblk1 = keep_ref[1, blk]

        def wcond(st):
            return st[2] > 0.0

        def wbody(st):
            act0, act1, _ = st
            new0 = jnp.where(_matvec(act0, supf0) > 0.5, 0.0, kblk0)
            new1 = jnp.where(_matvec(act1, supf1) > 0.5, 0.0, kblk1)
            changed = jnp.sum(jnp.abs(new0 - act0)) + jnp.sum(jnp.abs(new1 - act1))
            return new0, new1, changed

        act0, act1, _ = jax.lax.while_loop(
            wcond, wbody, (kblk0, kblk1, jnp.float32(1.0))
        )
        keep_ref[0, blk] = act0
        keep_ref[1, blk] = act1

        def suppress_at(w):
            s0 = jnp.where(_iou_tile(rows0, _col_boxes(colp_ref[0, w])) >= _IOU_THR, 1.0, 0.0)
            s1 = jnp.where(_iou_tile(rows1, _col_boxes(colp_ref[1, w])) >= _IOU_THR, 1.0, 0.0)
            sup0 = _matvec(act0, s0)
            sup1 = _matvec(act1, s1)
            keep_ref[0, w] = jnp.where(sup0 > 0.5, 0.0, keep_ref[0, w])
            keep_ref[1, w] = jnp.where(sup1 > 0.5, 0.0, keep_ref[1, w])

        nw = nb - blk - 1

        def w4_body(k, _):
            w = blk + 1 + 4 * k
            s0 = [
                jnp.where(
                    _iou_tile(rows0, _col_boxes(colp_ref[0, w + j])) >= _IOU_THR,
                    1.0, 0.0,
                )
                for j in range(4)
            ]
            s1 = [
                jnp.where(
                    _iou_tile(rows1, _col_boxes(colp_ref[1, w + j])) >= _IOU_THR,
                    1.0, 0.0,
                )
                for j in range(4)
            ]
            sup0 = _matvec(act0, jnp.concatenate(s0, axis=1))
            sup1 = _matvec(act1, jnp.concatenate(s1, axis=1))
            for j in range(4):
                sl = slice(j * _T, (j + 1) * _T)
                keep_ref[0, w + j] = jnp.where(sup0[:, sl] > 0.5, 0.0, keep_ref[0, w + j])
                keep_ref[1, w + j] = jnp.where(sup1[:, sl] > 0.5, 0.0, keep_ref[1, w + j])
            return ()

        jax.lax.fori_loop(0, nw // 4, w4_body, ())

        def tail_body(w, _):
            suppress_at(w)
            return ()

        jax.lax.fori_loop(blk + 1 + (nw // 4) * 4, nb, tail_body, ())
        return ()

    jax.lax.fori_loop(0, nb, blk_body, ())

    keep = keep_ref[...]
    outs_ref[...] = sc_ref[...] * keep
    outb_ref[...] = bc_ref[...] * keep


def kernel(bboxes, scores):
    bat, _, n = bboxes.shape
    npad = ((n + _T - 1) // _T) * _T
    nb = npad // _T

    order = jnp.argsort(-scores, axis=1)
    b = jnp.transpose(bboxes, (0, 2, 1))
    b_sorted = jnp.take_along_axis(b, order[:, :, None], axis=1)
    sc_sorted = jnp.take_along_axis(scores, order, axis=1)

    pad = npad - n
    bp = jnp.pad(b_sorted, ((0, 0), (0, pad), (0, 0)))
    scp = jnp.pad(sc_sorted, ((0, 0), (0, pad)))

    bt3 = bp.reshape(bat, nb, _T, 6)
    bc3 = jnp.transpose(bt3, (0, 1, 3, 2))
    sc3 = scp.reshape(bat, nb, 1, _T)

    outs, outb = pl.pallas_call(
        _nms_kernel,
        grid=(1,),
        in_specs=[
            pl.BlockSpec((bat, nb, 6, _T), lambda i: (0, 0, 0, 0)),
            pl.BlockSpec((bat, nb, _T, 6), lambda i: (0, 0, 0, 0)),
            pl.BlockSpec((bat, nb, 1, _T), lambda i: (0, 0, 0, 0)),
        ],
        out_specs=[
            pl.BlockSpec((bat, nb, 1, _T), lambda i: (0, 0, 0, 0)),
            pl.BlockSpec((bat, nb, 6, _T), lambda i: (0, 0, 0, 0)),
        ],
        out_shape=[
            jax.ShapeDtypeStruct((bat, nb, 1, _T), jnp.float32),
            jax.ShapeDtypeStruct((bat, nb, 6, _T), jnp.float32),
        ],
        scratch_shapes=[
            pltpu.VMEM((bat, nb, 1, _T), jnp.float32),
            pltpu.VMEM((bat, nb, 8, _T), jnp.float32),
        ],
    )(bc3, bt3, sc3)

    sel_scores = outs.reshape(bat, npad)[:, :n]
    sel_boxes = jnp.transpose(outb, (0, 1, 3, 2)).reshape(bat, npad, 6)[:, :n, :]
    return jnp.concatenate([sel_scores[:, :, None], sel_boxes], axis=2)
